# baseline (speedup 1.0000x reference)
import functools

import jax
import jax.numpy as jnp
from jax import lax
from jax.experimental import pallas as pl
from jax.experimental.pallas import tpu as pltpu
from jax.experimental.pallas import tpu_sc as plsc

DIM = 64
NC, NS, L = 2, 16, 16
NW = NC * NS
NBUF = 4
NOB = 2
LA = 2


@functools.lru_cache(maxsize=None)
def _make_transpose(vocab: int):
    n_full = vocab // (2 * L * 4)
    rem = vocab - n_full * 2 * L * 4
    trips = 2 * -(-n_full // (2 * NW))
    mesh = plsc.VectorSubcoreMesh(core_axis_name="c", subcore_axis_name="s")

    @functools.partial(
        pl.kernel,
        mesh=mesh,
        compiler_params=pltpu.CompilerParams(needs_layout_passes=False),
        out_type=jax.ShapeDtypeStruct((vocab // 2, 2 * DIM), jnp.float32),
        scratch_types=[
            pltpu.VMEM((2, DIM, 2 * DIM), jnp.float32),
            pltpu.VMEM((2, DIM, 2 * DIM), jnp.float32),
            pltpu.VMEM((DIM, L), jnp.int32),
        ]
        + [pltpu.SemaphoreType.DMA] * 2
        + [pltpu.SemaphoreType.DMA] * 2,
    )
    def k(tabt_hbm, tail_hbm, out_hbm, inb, outb, qtab, rs0, rs1, ws0, ws1):
        rsem = (rs0, rs1)
        wsem = (ws0, ws1)
        wid = lax.axis_index("s") * NC + lax.axis_index("c")
        vb0 = wid * trips
        last = jnp.int32(n_full - 1)

        def vb_of(t):
            return lax.min(vb0 + t, last)

        def read(t, i):
            pltpu.async_copy(
                tabt_hbm.at[:, pl.ds(vb_of(t) * 128, 128)], inb.at[i], rsem[i]
            )

        def transpose(i, nq, unroll):
            dvecs = [
                lax.iota(jnp.int32, L) + jnp.int32(kk * L) for kk in range(4)
            ]

            def qbody(qq, carry):
                for u in range(unroll):
                    q = qq * unroll + u
                    v0 = qtab[q, pl.ds(0, L)]
                    v1 = v0 + 1
                    for kk in range(8):
                        vv = v0 if kk < 4 else v1
                        outb[i, q, pl.ds(kk * L, L)] = plsc.load_gather(
                            inb.at[i], [dvecs[kk % 4], vv]
                        )
                return carry

            lax.fori_loop(0, nq // unroll, qbody, 0)

        for q in range(DIM):
            qtab[q, pl.ds(0, L)] = jnp.full((L,), 2 * q, jnp.int32)

        read(0, 0)
        read(1, 1)

        def outer(kk2, carry):
            for s in range(2):
                t = kk2 * 2 + s
                i = s
                pltpu.make_async_copy(
                    tabt_hbm.at[:, pl.ds(0, 128)], inb.at[i], rsem[i]
                ).wait()

                @pl.when(kk2 >= 1)
                def _():
                    pltpu.make_async_copy(
                        outb.at[i], out_hbm.at[pl.ds(0, DIM)], wsem[i]
                    ).wait()

                transpose(i, DIM, 4)
                pltpu.async_copy(
                    outb.at[i], out_hbm.at[pl.ds(vb_of(t) * DIM, DIM)], wsem[i]
                )

                @pl.when(kk2 < trips // 2 - 1)
                def _():
                    read(t + 2, i)

            return carry

        lax.fori_loop(0, trips // 2, outer, 0)
        for i in range(2):
            pltpu.make_async_copy(
                outb.at[i], out_hbm.at[pl.ds(0, DIM)], wsem[i]
            ).wait()

        if rem:
            @pl.when(wid == NW - 1)
            def _():
                pltpu.sync_copy(tail_hbm, outb.at[0, pl.ds(0, rem // 2)])
                pltpu.sync_copy(
                    outb.at[0, pl.ds(0, rem // 2)],
                    out_hbm.at[pl.ds(n_full * DIM, rem // 2)],
                )

    return k


@functools.lru_cache(maxsize=None)
def _make_sc_gather(batch: int, hist: int, vocab: int):
    assert batch % NW == 0
    bw = batch // NW
    nbg = bw // L
    mesh = plsc.VectorSubcoreMesh(core_axis_name="c", subcore_axis_name="s")

    @functools.partial(
        pl.kernel,
        mesh=mesh,
        compiler_params=pltpu.CompilerParams(needs_layout_passes=False),
        out_type=jax.ShapeDtypeStruct((batch, hist, DIM), jnp.float32),
        scratch_types=[
            pltpu.VMEM((hist, bw), jnp.int32),
            pltpu.VMEM((NBUF, bw), jnp.int32),
            pltpu.VMEM((NBUF, bw), jnp.int32),
            pltpu.VMEM((NBUF, bw, 2 * DIM), jnp.float32),
            pltpu.VMEM((NOB, bw, DIM), jnp.float32),
        ]
        + [pltpu.SemaphoreType.DMA] * NBUF
        + [pltpu.SemaphoreType.DMA] * NOB,
    )
    def k(idx_hbm, tab2_hbm, out_hbm, idx_v, pix_v, off_v, rows_v, outv, *sems):
        gsem = sems[:NBUF]
        wsem = sems[NBUF:]
        wid = lax.axis_index("s") * NC + lax.axis_index("c")
        base = wid * bw
        pltpu.sync_copy(idx_hbm.at[:, pl.ds(base, bw)], idx_v)

        def prep(h, i):
            for g in range(nbg):
                x = idx_v[h, pl.ds(g * L, L)]
                pix_v[i, pl.ds(g * L, L)] = lax.shift_right_logical(x, 1)
                off_v[i, pl.ds(g * L, L)] = lax.mul(
                    lax.bitwise_and(x, 1), jnp.int32(DIM)
                )

        def gather(i):
            pltpu.async_copy(tab2_hbm.at[pix_v.at[i]], rows_v.at[i], gsem[i])

        for h in range(LA):
            prep(h, h)
            gather(h)

        def slot(h, i, o, first, last):
            pltpu.make_async_copy(
                tab2_hbm.at[pix_v.at[i]], rows_v.at[i], gsem[i]
            ).wait()
            if not last:
                j = (i + LA) % NBUF
                prep(h + LA, j)
                gather(j)
            if not first:
                pltpu.make_async_copy(
                    outv.at[o], out_hbm.at[pl.ds(base, bw), 0], wsem[o]
                ).wait()
            def bbody(bg, carry):
                offv = off_v[i, pl.ds(bg * L, L)]
                for u in range(L):
                    b = bg * L + u
                    off = offv[u]
                    for q in range(DIM // L):
                        outv[o, b, pl.ds(q * L, L)] = rows_v[
                            i, b, pl.ds(off + q * L, L)
                        ]
                return carry

            lax.fori_loop(0, bw // L, bbody, 0)
            pltpu.async_copy(
                outv.at[o], out_hbm.at[pl.ds(base, bw), h], wsem[o]
            )

        n_main = hist - LA
        assert n_main % NBUF == 0

        def outer(kk, carry):
            h0 = kk * NBUF
            for s in range(NBUF):
                slot(h0 + s, s, s % NOB, first=False, last=False)
            return carry

        for s in range(NBUF):
            slot(s, s, s % NOB, first=(s < NOB), last=False)
        lax.fori_loop(1, n_main // NBUF, outer, 0)
        for t in range(LA):
            h = n_main + t
            slot(h, h % NBUF, h % NOB, first=False, last=True)

        for t in range(NOB):
            o = (hist - 1 - t) % NOB
            pltpu.make_async_copy(
                outv.at[o], out_hbm.at[pl.ds(base, bw), 0], wsem[o]
            ).wait()

    return k


def kernel(inputs, table):
    batch, hist = inputs.shape
    vocab = table.shape[0]
    n_full = vocab // 128
    tail = table[n_full * 128 :].reshape(-1, 2 * DIM)
    table2 = _make_transpose(vocab)(table.T, tail)
    return _make_sc_gather(batch, hist, vocab)(inputs.T, table2)


# speedup vs baseline: 1.0012x; 1.0012x over previous
import functools

import jax
import jax.numpy as jnp
from jax import lax
from jax.experimental import pallas as pl
from jax.experimental.pallas import tpu as pltpu
from jax.experimental.pallas import tpu_sc as plsc

DIM = 64
NC, NS, L = 2, 16, 16
NW = NC * NS
NBUF = 4
NOB = 2
LA = 2


@functools.lru_cache(maxsize=None)
def _make_transpose(vocab: int):
    n_full = vocab // (2 * L * 4)
    rem = vocab - n_full * 2 * L * 4
    trips = 2 * -(-n_full // (2 * NW))
    mesh = plsc.VectorSubcoreMesh(core_axis_name="c", subcore_axis_name="s")

    @functools.partial(
        pl.kernel,
        mesh=mesh,
        compiler_params=pltpu.CompilerParams(
            needs_layout_passes=False, disable_bounds_checks=True
        ),
        out_type=jax.ShapeDtypeStruct((vocab // 2, 2 * DIM), jnp.float32),
        scratch_types=[
            pltpu.VMEM((2, DIM, 2 * DIM), jnp.float32),
            pltpu.VMEM((2, DIM, 2 * DIM), jnp.float32),
            pltpu.VMEM((DIM, L), jnp.int32),
        ]
        + [pltpu.SemaphoreType.DMA] * 2
        + [pltpu.SemaphoreType.DMA] * 2,
    )
    def k(tabt_hbm, tail_hbm, out_hbm, inb, outb, qtab, rs0, rs1, ws0, ws1):
        rsem = (rs0, rs1)
        wsem = (ws0, ws1)
        wid = lax.axis_index("s") * NC + lax.axis_index("c")
        vb0 = wid * trips
        last = jnp.int32(n_full - 1)

        def vb_of(t):
            return lax.min(vb0 + t, last)

        def read(t, i):
            pltpu.async_copy(
                tabt_hbm.at[:, pl.ds(vb_of(t) * 128, 128)], inb.at[i], rsem[i]
            )

        def transpose(i, nq, unroll):
            dvecs = [
                lax.iota(jnp.int32, L) + jnp.int32(kk * L) for kk in range(4)
            ]

            def qbody(qq, carry):
                for u in range(unroll):
                    q = qq * unroll + u
                    v0 = qtab[q, pl.ds(0, L)]
                    v1 = v0 + 1
                    for kk in range(8):
                        vv = v0 if kk < 4 else v1
                        outb[i, q, pl.ds(kk * L, L)] = plsc.load_gather(
                            inb.at[i], [dvecs[kk % 4], vv]
                        )
                return carry

            lax.fori_loop(0, nq // unroll, qbody, 0)

        for q in range(DIM):
            qtab[q, pl.ds(0, L)] = jnp.full((L,), 2 * q, jnp.int32)

        read(0, 0)
        read(1, 1)

        def outer(kk2, carry):
            for s in range(2):
                t = kk2 * 2 + s
                i = s
                pltpu.make_async_copy(
                    tabt_hbm.at[:, pl.ds(0, 128)], inb.at[i], rsem[i]
                ).wait()

                @pl.when(kk2 >= 1)
                def _():
                    pltpu.make_async_copy(
                        outb.at[i], out_hbm.at[pl.ds(0, DIM)], wsem[i]
                    ).wait()

                transpose(i, DIM, 4)
                pltpu.async_copy(
                    outb.at[i], out_hbm.at[pl.ds(vb_of(t) * DIM, DIM)], wsem[i]
                )

                @pl.when(kk2 < trips // 2 - 1)
                def _():
                    read(t + 2, i)

            return carry

        lax.fori_loop(0, trips // 2, outer, 0)
        for i in range(2):
            pltpu.make_async_copy(
                outb.at[i], out_hbm.at[pl.ds(0, DIM)], wsem[i]
            ).wait()

        if rem:
            @pl.when(wid == NW - 1)
            def _():
                pltpu.sync_copy(tail_hbm, outb.at[0, pl.ds(0, rem // 2)])
                pltpu.sync_copy(
                    outb.at[0, pl.ds(0, rem // 2)],
                    out_hbm.at[pl.ds(n_full * DIM, rem // 2)],
                )

    return k


@functools.lru_cache(maxsize=None)
def _make_sc_gather(batch: int, hist: int, vocab: int):
    assert batch % NW == 0
    bw = batch // NW
    nbg = bw // L
    mesh = plsc.VectorSubcoreMesh(core_axis_name="c", subcore_axis_name="s")

    @functools.partial(
        pl.kernel,
        mesh=mesh,
        compiler_params=pltpu.CompilerParams(needs_layout_passes=False),
        out_type=jax.ShapeDtypeStruct((batch, hist, DIM), jnp.float32),
        scratch_types=[
            pltpu.VMEM((hist, bw), jnp.int32),
            pltpu.VMEM((NBUF, bw), jnp.int32),
            pltpu.VMEM((NBUF, bw), jnp.int32),
            pltpu.VMEM((NBUF, bw, 2 * DIM), jnp.float32),
            pltpu.VMEM((NOB, bw, DIM), jnp.float32),
        ]
        + [pltpu.SemaphoreType.DMA] * NBUF
        + [pltpu.SemaphoreType.DMA] * NOB,
    )
    def k(idx_hbm, tab2_hbm, out_hbm, idx_v, pix_v, off_v, rows_v, outv, *sems):
        gsem = sems[:NBUF]
        wsem = sems[NBUF:]
        wid = lax.axis_index("s") * NC + lax.axis_index("c")
        base = wid * bw
        pltpu.sync_copy(idx_hbm.at[:, pl.ds(base, bw)], idx_v)

        def prep(h, i):
            for g in range(nbg):
                x = idx_v[h, pl.ds(g * L, L)]
                pix_v[i, pl.ds(g * L, L)] = lax.shift_right_logical(x, 1)
                off_v[i, pl.ds(g * L, L)] = lax.mul(
                    lax.bitwise_and(x, 1), jnp.int32(DIM)
                )

        def gather(i):
            pltpu.async_copy(tab2_hbm.at[pix_v.at[i]], rows_v.at[i], gsem[i])

        for h in range(LA):
            prep(h, h)
            gather(h)

        def slot(h, i, o, first, last):
            pltpu.make_async_copy(
                tab2_hbm.at[pix_v.at[i]], rows_v.at[i], gsem[i]
            ).wait()
            if not last:
                j = (i + LA) % NBUF
                prep(h + LA, j)
                gather(j)
            if not first:
                pltpu.make_async_copy(
                    outv.at[o], out_hbm.at[pl.ds(base, bw), 0], wsem[o]
                ).wait()
            def bbody(bg, carry):
                offv = off_v[i, pl.ds(bg * L, L)]
                for u in range(L):
                    b = bg * L + u
                    off = offv[u]
                    for q in range(DIM // L):
                        outv[o, b, pl.ds(q * L, L)] = rows_v[
                            i, b, pl.ds(off + q * L, L)
                        ]
                return carry

            lax.fori_loop(0, bw // L, bbody, 0)
            pltpu.async_copy(
                outv.at[o], out_hbm.at[pl.ds(base, bw), h], wsem[o]
            )

        n_main = hist - LA
        assert n_main % NBUF == 0

        def outer(kk, carry):
            h0 = kk * NBUF
            for s in range(NBUF):
                slot(h0 + s, s, s % NOB, first=False, last=False)
            return carry

        for s in range(NBUF):
            slot(s, s, s % NOB, first=(s < NOB), last=False)
        lax.fori_loop(1, n_main // NBUF, outer, 0)
        for t in range(LA):
            h = n_main + t
            slot(h, h % NBUF, h % NOB, first=False, last=True)

        for t in range(NOB):
            o = (hist - 1 - t) % NOB
            pltpu.make_async_copy(
                outv.at[o], out_hbm.at[pl.ds(base, bw), 0], wsem[o]
            ).wait()

    return k


def kernel(inputs, table):
    batch, hist = inputs.shape
    vocab = table.shape[0]
    n_full = vocab // 128
    tail = table[n_full * 128 :].reshape(-1, 2 * DIM)
    table2 = _make_transpose(vocab)(table.T, tail)
    return _make_sc_gather(batch, hist, vocab)(inputs.T, table2)


# speedup vs baseline: 2.1041x; 2.1016x over previous
import functools

import jax
import jax.numpy as jnp
from jax import lax
from jax.experimental import pallas as pl
from jax.experimental.pallas import tpu as pltpu
from jax.experimental.pallas import tpu_sc as plsc

DIM = 64
NC, NS, L = 2, 16, 16
NW = NC * NS
NBUF = 4
LA = 2


@functools.lru_cache(maxsize=None)
def _make_sc_gather(batch: int, hist: int, vocab: int):
    assert batch % NW == 0
    bw = batch // NW
    mesh = plsc.VectorSubcoreMesh(core_axis_name="c", subcore_axis_name="s")

    @functools.partial(
        pl.kernel,
        mesh=mesh,
        compiler_params=pltpu.CompilerParams(needs_layout_passes=False),
        out_type=jax.ShapeDtypeStruct((batch, hist, 2 * DIM), jnp.float32),
        scratch_types=[
            pltpu.VMEM((hist, bw), jnp.int32),
            pltpu.VMEM((NBUF, bw, 2 * DIM), jnp.float32),
        ]
        + [pltpu.SemaphoreType.DMA] * NBUF
        + [pltpu.SemaphoreType.DMA] * NBUF,
    )
    def k(idx_hbm, tab_hbm, out_hbm, idx_v, rows_v, *sems):
        gsem = sems[:NBUF]
        wsem = sems[NBUF:]
        wid = lax.axis_index("s") * NC + lax.axis_index("c")
        base = wid * bw
        pltpu.sync_copy(idx_hbm.at[:, pl.ds(base, bw)], idx_v)

        def gather(h, i):
            pltpu.async_copy(tab_hbm.at[idx_v.at[h]], rows_v.at[i], gsem[i])

        for h in range(LA):
            gather(h, h)

        def slot(h, i, first, last):
            pltpu.make_async_copy(
                tab_hbm.at[idx_v.at[h]], rows_v.at[i], gsem[i]
            ).wait()
            pltpu.sync_copy(rows_v.at[i], out_hbm.at[pl.ds(base, bw), h])
            if not last:
                gather(h + LA, (i + LA) % NBUF)

        n_main = hist - LA
        assert n_main % NBUF == 0

        def outer(kk, carry):
            h0 = kk * NBUF
            for s in range(NBUF):
                slot(h0 + s, s, first=False, last=False)
            return carry

        for s in range(NBUF):
            slot(s, s, first=(s < LA), last=False)
        lax.fori_loop(1, n_main // NBUF, outer, 0)
        for t in range(LA):
            h = n_main + t
            slot(h, h % NBUF, first=False, last=True)

    return k


def kernel(inputs, table):
    batch, hist = inputs.shape
    vocab = table.shape[0]
    table_wide = jnp.concatenate([table, table], axis=1)
    out_wide = _make_sc_gather(batch, hist, vocab)(inputs.T, table_wide)
    return out_wide[:, :, :DIM]


# speedup vs baseline: 2.2190x; 1.0546x over previous
import functools

import jax
import jax.numpy as jnp
from jax import lax
from jax.experimental import pallas as pl
from jax.experimental.pallas import tpu as pltpu
from jax.experimental.pallas import tpu_sc as plsc

DIM = 64
NC, NS = 2, 16
NW = NC * NS
NBUF = 8
LOOKAHEAD = 4


@functools.lru_cache(maxsize=None)
def _make_sc_gather(batch: int, hist: int, vocab: int):
    assert batch % NW == 0
    bw = batch // NW
    mesh = plsc.VectorSubcoreMesh(core_axis_name="c", subcore_axis_name="s")

    @functools.partial(
        pl.kernel,
        mesh=mesh,
        compiler_params=pltpu.CompilerParams(use_tc_tiling_on_sc=False),
        out_type=jax.ShapeDtypeStruct((batch, hist, DIM), jnp.float32),
        scratch_types=[
            pltpu.VMEM((hist, bw), jnp.int32),
            pltpu.VMEM((NBUF, bw, DIM), jnp.float32),
        ]
        + [pltpu.SemaphoreType.DMA] * NBUF
        + [pltpu.SemaphoreType.DMA] * NBUF,
    )
    def k(idx_hbm, table_hbm, out_hbm, idx_v, rows_v, *sems):
        gsem = sems[:NBUF]
        wsem = sems[NBUF:]
        wid = lax.axis_index("s") * NC + lax.axis_index("c")
        base = wid * bw
        pltpu.sync_copy(idx_hbm.at[:, pl.ds(base, bw)], idx_v)

        def gather(h, i):
            pltpu.async_copy(table_hbm.at[idx_v.at[h]], rows_v.at[i], gsem[i])

        for h in range(LOOKAHEAD):
            gather(h, h)

        for h in range(hist):
            i = h % NBUF
            pltpu.make_async_copy(
                table_hbm.at[idx_v.at[h]], rows_v.at[i], gsem[i]
            ).wait()
            pltpu.async_copy(
                rows_v.at[i], out_hbm.at[pl.ds(base, bw), h], wsem[i]
            )
            g = h + LOOKAHEAD
            if g < hist:
                j = g % NBUF
                if g >= NBUF:
                    pltpu.make_async_copy(
                        rows_v.at[j], out_hbm.at[pl.ds(base, bw), 0], wsem[j]
                    ).wait()
                gather(g, j)

        for t in range(min(NBUF, hist)):
            i = (hist - 1 - t) % NBUF
            pltpu.make_async_copy(
                rows_v.at[i], out_hbm.at[pl.ds(base, bw), 0], wsem[i]
            ).wait()

    return k


def kernel(inputs, table):
    batch, hist = inputs.shape
    return _make_sc_gather(batch, hist, table.shape[0])(inputs.T, table)
